# baseline (speedup 1.0000x reference)
import functools
import math

import jax
import jax.numpy as jnp
from jax import lax
from jax.experimental import pallas as pl
from jax.experimental.pallas import tpu as pltpu
from jax.experimental.pallas import tpu_sc as plsc

N = 10000
E = 320000
D = 128
DH = D // 2

NC = 2
NS = 16
NW = NC * NS

L = 16

CH = 125
CHP = 128
CN = (E // CH) // NS
EWP = CN * CHP
ROWS_PER_TILE = N // NS
WB = ROWS_PER_TILE // CH
CPC = CHP // L
ND = N + L


def _make_sc_kernel():
    mesh = plsc.VectorSubcoreMesh(core_axis_name="c", subcore_axis_name="s")

    @functools.partial(
        pl.kernel,
        out_type=[
            jax.ShapeDtypeStruct((NC, N, DH), jnp.float32),
            jax.ShapeDtypeStruct((NS, N), jnp.float32),
        ],
        mesh=mesh,
        compiler_params=pltpu.CompilerParams(use_tc_tiling_on_sc=False,
                                             needs_layout_passes=False),
        scratch_types=[
            pltpu.VMEM((EWP,), jnp.int32),
            pltpu.VMEM((EWP,), jnp.int32),
            pltpu.VMEM((4, CHP, DH), jnp.float32),
            pltpu.VMEM((ND,), jnp.float32),
            pltpu.VMEM_SHARED((ND, DH), jnp.float32),
            pltpu.SemaphoreType.DMA,
            pltpu.SemaphoreType.DMA,
            pltpu.SemaphoreType.DMA,
            pltpu.SemaphoreType.DMA,
            pltpu.SemaphoreType.DMA,
            pltpu.SemaphoreType.DMA,
            pltpu.SemaphoreType.DMA,
            pltpu.SemaphoreType.DMA,
            pltpu.SemaphoreType.DMA,
        ],
    )
    def sc_kernel(x2_hbm, srce, srco, dst1d, agg_out, deg_out,
                  src_v, dst_v, rows_v, deg_v, agg_sh,
                  g0, g1, g2, g3, s0, s1, s2, s3, stsem):
        cid = lax.axis_index("c")
        sid = lax.axis_index("s")
        base = sid * EWP
        rbase = sid * ROWS_PER_TILE
        zvec = jnp.zeros((L,), jnp.float32)
        ovec = jnp.ones((L,), jnp.float32)
        gsems = [g0, g1, g2, g3]
        ssems = [s0, s1, s2, s3]

        @pl.when(cid == 0)
        def _():
            pltpu.async_copy(srce.at[pl.ds(base, EWP)], src_v, stsem)

        @pl.when(cid == 1)
        def _():
            pltpu.async_copy(srco.at[pl.ds(base, EWP)], src_v, stsem)

        pltpu.async_copy(dst1d.at[pl.ds(base, EWP)], dst_v, s2)

        def fill_rows(i, _):
            for k in range(DH // L):
                rows_v[0, i, pl.ds(k * L, L)] = zvec
            return 0
        lax.fori_loop(0, CH, fill_rows, 0)

        def fill_deg(i, _):
            deg_v[pl.ds(i * L, L)] = zvec
            return 0
        lax.fori_loop(0, ND // L, fill_deg, 0)

        for t in range(WB):
            pltpu.sync_copy(rows_v.at[0, pl.ds(0, CH)],
                            agg_sh.at[pl.ds(rbase + t * CH, CH)])
        pltpu.make_async_copy(srce.at[pl.ds(base, EWP)], src_v, stsem).wait()
        pltpu.make_async_copy(dst1d.at[pl.ds(base, EWP)], dst_v, s2).wait()
        plsc.subcore_barrier()

        def gather(j, k):
            pltpu.async_copy(x2_hbm.at[src_v.at[pl.ds(j * CHP, CHP)]],
                             rows_v.at[k], gsems[k])

        def wait_gather(j, k):
            pltpu.make_async_copy(x2_hbm.at[src_v.at[pl.ds(j * CHP, CHP)]],
                                  rows_v.at[k], gsems[k]).wait()

        def scatter(j, k):
            pltpu.async_copy(rows_v.at[k],
                             agg_sh.at[dst_v.at[pl.ds(j * CHP, CHP)]],
                             ssems[k], add=True)

        def wait_scatter(j, k):
            pltpu.make_async_copy(rows_v.at[k],
                                  agg_sh.at[dst_v.at[pl.ds(j * CHP, CHP)]],
                                  ssems[k]).wait()

        def count_vecs(s0_, n_):
            def step(s, _):
                idx = dst_v[pl.ds(s * L, L)]
                plsc.addupdate_scatter(deg_v, [idx], ovec)
                return 0
            lax.fori_loop(s0_, s0_ + n_, step, 0)

        def quad(t, _):
            for q in range(4):
                j = t * 4 + q
                wait_gather(j, q)
                scatter(j, q)

                @pl.when(cid == 0)
                def _():
                    count_vecs(j * CPC, CPC)

                @pl.when(j >= 2)
                def _():
                    wait_scatter(j - 2, (q + 2) % 4)

                @pl.when(j + 2 < CN)
                def _():
                    gather(j + 2, (q + 2) % 4)
            return 0

        gather(0, 0)
        gather(1, 1)
        lax.fori_loop(0, CN // 4, quad, 0)
        wait_scatter(CN - 2, 2)
        wait_scatter(CN - 1, 3)

        plsc.subcore_barrier()

        for t in range(WB):
            pltpu.sync_copy(agg_sh.at[pl.ds(rbase + t * CH, CH)],
                            agg_out.at[cid, pl.ds(rbase + t * CH, CH)])

        @pl.when(cid == 0)
        def _():
            pltpu.sync_copy(deg_v.at[pl.ds(0, N)], deg_out.at[sid])

    return sc_kernel


_sc_kernel = _make_sc_kernel()


def _tc_body(agg_ref, deg_ref, x_ref, w_ref, b_ref, g_ref, bt_ref, out_ref):
    a = jnp.concatenate([agg_ref[0], agg_ref[1]], axis=1)
    dg = jnp.sum(deg_ref[...], axis=0)[:, None]
    dg = jnp.maximum(dg, 1.0)
    an = a / dg
    h = lax.dot_general(an, w_ref[...], (((1,), (0,)), ((), ())),
                        preferred_element_type=jnp.float32,
                        precision=lax.Precision.HIGHEST)
    o = h + b_ref[...] + x_ref[...]
    mu = jnp.mean(o, axis=1, keepdims=True)
    c = o - mu
    var = jnp.mean(c * c, axis=1, keepdims=True)
    y = c * lax.rsqrt(var + 1e-5) * g_ref[...] + bt_ref[...]
    out_ref[...] = 0.5 * y * (1.0 + lax.erf(y * (1.0 / math.sqrt(2.0))))


def _tc_tail(agg_p, deg_p, x, W, b, g, bt):
    return pl.pallas_call(
        _tc_body,
        out_shape=jax.ShapeDtypeStruct((N, D), jnp.float32),
    )(agg_p, deg_p, x, W, b, g, bt)


@jax.jit
def kernel(x, edge_index, W, b, ln_gamma, ln_beta):
    nchunk = E // CH
    src2 = edge_index[0] * 2
    spad = jnp.zeros((nchunk, CHP - CH), jnp.int32)
    dpad = jnp.broadcast_to(
        jnp.arange(N, N + CHP - CH, dtype=jnp.int32), (nchunk, CHP - CH))
    srce = jnp.concatenate(
        [src2.reshape(nchunk, CH), spad], axis=1).reshape(-1)
    srco = jnp.concatenate(
        [(src2 + 1).reshape(nchunk, CH), spad], axis=1).reshape(-1)
    dstp = jnp.concatenate(
        [edge_index[1].reshape(nchunk, CH), dpad], axis=1).reshape(-1)
    x2 = x.reshape(2 * N, DH)
    agg_p, deg_p = _sc_kernel(x2, srce, srco, dstp)
    return _tc_tail(agg_p, deg_p, x, W,
                    b.reshape(1, D), ln_gamma.reshape(1, D),
                    ln_beta.reshape(1, D))


# speedup vs baseline: 2.2481x; 2.2481x over previous
import functools
import math

import jax
import jax.numpy as jnp
from jax import lax
from jax.experimental import pallas as pl
from jax.experimental.pallas import tpu as pltpu
from jax.experimental.pallas import tpu_sc as plsc

N = 10000
E = 320000
D = 128
DH = D // 2

NC = 2
NS = 16
NW = NC * NS

L = 16

CH = 125
CN = (E // CH) // NS
EW = E // NW
ROWS_PER_TILE = N // NS
WB = ROWS_PER_TILE // CH


def _make_sc_kernel():
    mesh = plsc.VectorSubcoreMesh(core_axis_name="c", subcore_axis_name="s")

    @functools.partial(
        pl.kernel,
        out_type=[
            jax.ShapeDtypeStruct((NC, N, DH), jnp.float32),
            jax.ShapeDtypeStruct((NW, N), jnp.float32),
        ],
        mesh=mesh,
        compiler_params=pltpu.CompilerParams(use_tc_tiling_on_sc=False,
                                             needs_layout_passes=False),
        scratch_types=[
            pltpu.VMEM((CN, CH), jnp.int32),
            pltpu.VMEM((CN, CH), jnp.int32),
            pltpu.VMEM((EW,), jnp.int32),
            pltpu.VMEM((2, CH, DH), jnp.float32),
            pltpu.VMEM((N,), jnp.float32),
            pltpu.VMEM_SHARED((N, DH), jnp.float32),
            pltpu.SemaphoreType.DMA,
            pltpu.SemaphoreType.DMA,
            pltpu.SemaphoreType.DMA,
            pltpu.SemaphoreType.DMA,
            pltpu.SemaphoreType.DMA,
            pltpu.SemaphoreType.DMA,
            pltpu.SemaphoreType.DMA,
            pltpu.SemaphoreType.DMA,
            pltpu.SemaphoreType.DMA,
        ],
    )
    def sc_kernel(x2_hbm, srce2d, srco2d, dst2d, dst1d, agg_out, deg_out,
                  src_v, dst_v, dstf_v, rows_v, deg_v, agg_sh,
                  g0, g1, g2, g3, s0, s1, s2, s3, stsem):
        cid = lax.axis_index("c")
        sid = lax.axis_index("s")
        w = cid * NS + sid
        base = sid * CN
        rbase = sid * ROWS_PER_TILE
        zvec = jnp.zeros((L,), jnp.float32)
        ovec = jnp.ones((L,), jnp.float32)
        gsems = [g0, g1, g2, g3]
        ssems = [s0, s1, s2, s3]

        @pl.when(cid == 0)
        def _():
            pltpu.async_copy(srce2d.at[pl.ds(base, CN)], src_v, stsem)

        @pl.when(cid == 1)
        def _():
            pltpu.async_copy(srco2d.at[pl.ds(base, CN)], src_v, stsem)

        pltpu.async_copy(dst2d.at[pl.ds(base, CN)], dst_v, s2)
        pltpu.async_copy(dst1d.at[pl.ds(w * EW, EW)], dstf_v, s3)

        def fill_rows(i, _):
            for k in range(DH // L):
                rows_v[0, i, pl.ds(k * L, L)] = zvec
            return 0
        lax.fori_loop(0, CH, fill_rows, 0)

        def fill_deg(i, _):
            deg_v[pl.ds(i * L, L)] = zvec
            return 0
        lax.fori_loop(0, N // L, fill_deg, 0)

        for t in range(WB):
            pltpu.sync_copy(rows_v.at[0],
                            agg_sh.at[pl.ds(rbase + t * CH, CH)])
        pltpu.make_async_copy(srce2d.at[pl.ds(base, CN)], src_v, stsem).wait()
        pltpu.make_async_copy(dst2d.at[pl.ds(base, CN)], dst_v, s2).wait()
        pltpu.make_async_copy(dst1d.at[pl.ds(w * EW, EW)], dstf_v, s3).wait()
        plsc.subcore_barrier()

        def gather(j, k):
            pltpu.async_copy(x2_hbm.at[src_v.at[j]], rows_v.at[k], gsems[k])

        def wait_gather(j, k):
            pltpu.make_async_copy(x2_hbm.at[src_v.at[j]], rows_v.at[k],
                                  gsems[k]).wait()

        def scatter(j, k):
            pltpu.async_copy(rows_v.at[k], agg_sh.at[dst_v.at[j]], ssems[k],
                             add=True)

        def wait_scatter(j, k):
            pltpu.make_async_copy(rows_v.at[k], agg_sh.at[dst_v.at[j]],
                                  ssems[k]).wait()

        def quad(t, _):
            for q in range(4):
                j = t * 4 + q
                k = q % 2
                wait_gather(j, k)
                scatter(j, k)
                wait_scatter(j, k)

                @pl.when(j + 2 < CN)
                def _():
                    gather(j + 2, k)
            return 0

        gather(0, 0)
        gather(1, 1)
        lax.fori_loop(0, CN // 4, quad, 0)

        def count(i, _):
            idx = dstf_v[pl.ds(i * L, L)]
            plsc.addupdate_scatter(deg_v, [idx], ovec)
            return 0
        lax.fori_loop(0, EW // L, count, 0)

        plsc.subcore_barrier()

        for t in range(WB):
            pltpu.sync_copy(agg_sh.at[pl.ds(rbase + t * CH, CH)],
                            agg_out.at[cid, pl.ds(rbase + t * CH, CH)])
        pltpu.sync_copy(deg_v, deg_out.at[w])

    return sc_kernel


_sc_kernel = _make_sc_kernel()


def _tc_body(agg_ref, deg_ref, x_ref, w_ref, b_ref, g_ref, bt_ref, out_ref):
    a = jnp.concatenate([agg_ref[0], agg_ref[1]], axis=1)
    dg = jnp.sum(deg_ref[...], axis=0)[:, None]
    dg = jnp.maximum(dg, 1.0)
    an = a / dg
    h = lax.dot_general(an, w_ref[...], (((1,), (0,)), ((), ())),
                        preferred_element_type=jnp.float32,
                        precision=lax.Precision.HIGHEST)
    o = h + b_ref[...] + x_ref[...]
    mu = jnp.mean(o, axis=1, keepdims=True)
    c = o - mu
    var = jnp.mean(c * c, axis=1, keepdims=True)
    y = c * lax.rsqrt(var + 1e-5) * g_ref[...] + bt_ref[...]
    out_ref[...] = 0.5 * y * (1.0 + lax.erf(y * (1.0 / math.sqrt(2.0))))


def _tc_tail(agg_p, deg_p, x, W, b, g, bt):
    return pl.pallas_call(
        _tc_body,
        out_shape=jax.ShapeDtypeStruct((N, D), jnp.float32),
    )(agg_p, deg_p, x, W, b, g, bt)


@jax.jit
def kernel(x, edge_index, W, b, ln_gamma, ln_beta):
    nchunk = E // CH
    src2 = edge_index[0] * 2
    src2d_even = src2.reshape(nchunk, CH)
    src2d_odd = (src2 + 1).reshape(nchunk, CH)
    dst2d = edge_index[1].reshape(nchunk, CH)
    x2 = x.reshape(2 * N, DH)
    agg_p, deg_p = _sc_kernel(x2, src2d_even, src2d_odd, dst2d,
                              edge_index[1])
    return _tc_tail(agg_p, deg_p, x, W,
                    b.reshape(1, D), ln_gamma.reshape(1, D),
                    ln_beta.reshape(1, D))
